# baseline (speedup 1.0000x reference)
import jax
import jax.numpy as jnp
from jax.experimental import pallas as pl
from jax.experimental.pallas import tpu as pltpu

T = 16384
E = 64
D = 64
WR = 32
W = 128
BT = 128
N = T // BT
G = 4
NG = N // G


def _body(x_hbm, o_hbm, vbuf, obuf, gsem, wsem):
    ridx = 7 - jax.lax.broadcasted_iota(jnp.int32, (BT, 8, W), 1)

    def copy_in(g, s, b):
        return pltpu.make_async_copy(
            x_hbm.at[pl.ds((g * G + b) * BT, BT)], vbuf.at[s, b],
            gsem.at[s, b])

    def copy_out(g, s, b):
        return pltpu.make_async_copy(
            obuf.at[s, b], o_hbm.at[pl.ds((g * G + b) * BT, BT)],
            wsem.at[s, b])

    def flips(s):
        for b in range(G):
            for k in range(WR // 8):
                seg = vbuf[s, b, :, 8 * k:8 * (k + 1), :]
                seg = jnp.take_along_axis(seg, ridx, axis=1)
                seg = pltpu.roll(seg, W // 2, 2)
                obuf[s, b, :, WR - 8 * (k + 1):WR - 8 * k, :] = seg

    def stage(g, s, drain, refill):
        for b in range(G):
            copy_in(g, s, b).wait()
        if drain:
            for b in range(G):
                copy_out(g - 2, s, b).wait()
        flips(s)
        for b in range(G):
            copy_out(g, s, b).start()
            if refill:
                copy_in(g + 2, s, b).start()

    for b in range(G):
        copy_in(0, 0, b).start()
    for b in range(G):
        copy_in(1, 1, b).start()
    stage(0, 0, drain=False, refill=True)
    stage(1, 1, drain=False, refill=True)

    def super_group(sg, carry):
        g = 2 * sg
        stage(g, 0, drain=True, refill=True)
        stage(g + 1, 1, drain=True, refill=True)
        return carry

    jax.lax.fori_loop(1, NG // 2 - 1, super_group, 0)

    stage(NG - 2, 0, drain=True, refill=False)
    stage(NG - 1, 1, drain=True, refill=False)
    for s, g in ((0, NG - 2), (1, NG - 1)):
        for b in range(G):
            copy_out(g, s, b).wait()


def kernel(x):
    x4 = x.reshape(T, WR, W)
    y4 = pl.pallas_call(
        _body,
        in_specs=[pl.BlockSpec(memory_space=pltpu.HBM)],
        out_specs=pl.BlockSpec(memory_space=pltpu.HBM),
        out_shape=jax.ShapeDtypeStruct((T, WR, W), jnp.float32),
        scratch_shapes=[
            pltpu.VMEM((2, G, BT, WR, W), jnp.float32),
            pltpu.VMEM((2, G, BT, WR, W), jnp.float32),
            pltpu.SemaphoreType.DMA((2, G)),
            pltpu.SemaphoreType.DMA((2, G)),
        ],
    )(x4)
    return y4.reshape(T, E, D)
